# baseline (speedup 1.0000x reference)
import functools

import jax
import jax.numpy as jnp
from jax import lax
from jax.experimental import pallas as pl
from jax.experimental.pallas import tpu as pltpu
from jax.experimental.pallas import tpu_sc as plsc

B, T, D, F = 16, 512, 256, 256
ML = 4096
TP = T + 1
TPP = 528
L = 16
NC, NS = 2, 16
NW = NC * NS
POS_PER_W = ML * B // NW
CHUNK = 64
NCHUNK = POS_PER_W // CHUNK
SRC_ROWS = 128
GR = 16


def _expand_body(xflat, dur, out, durv, cumv, zv, idxb, srcbs, outbs,
                 rsems, wsems, fsems):
    cid = lax.axis_index("c")
    sid = lax.axis_index("s")
    b = sid
    half = cid

    pltpu.sync_copy(dur.at[b], durv.at[pl.ds(0, T)])
    durv[pl.ds(T, L)] = jnp.ones((L,), jnp.int32)

    carry = jnp.int32(0)
    for j in range(T // L):
        d = durv[pl.ds(j * L, L)]
        cs = plsc.cumsum(d) + carry
        cumv[pl.ds(j * L, L)] = cs
        carry = jnp.max(cs)

    def _zero(i, c):
        zv[pl.ds(i * L, L)] = jnp.zeros((L,), jnp.int32)
        return c
    lax.fori_loop(0, ML // L, _zero, 0)

    for j in range(T // L):
        cs = cumv[pl.ds(j * L, L)]
        dnext = durv[pl.ds(j * L + 1, L)]
        val = lax.iota(jnp.int32, L) + (j * L + 1)
        m = (dnext != 0) & (cs < ML)
        plsc.store_scatter(zv, [jnp.clip(cs, 0, ML - 1)], val, mask=m)

    base = b * TPP

    def _scan_chunk(c, run):
        for v in range(CHUNK // L):
            zz = zv[pl.ds(c * CHUNK + v * L, L)]
            r = jnp.maximum(plsc.cummax(zz), run)
            idxb[c, pl.ds(v * L, L)] = r + base
            run = jnp.max(r)
        return run
    lax.fori_loop(0, ML // CHUNK, _scan_chunk, jnp.int32(0))

    def _pair(t, carry):
        for sub in range(2):
            m = 2 * t + sub
            c = 2 * m + half
            v0 = idxb[c, pl.ds(0, L)]
            vl = idxb[c, pl.ds(CHUNK - L, L)]
            lo = jnp.min(v0) - base
            hi = jnp.max(vl) - base
            glo = (jnp.clip(lo, 0, TP - SRC_ROWS) // GR) * GR
            ok = (hi - glo) <= (SRC_ROWS - 1)
            srcb = srcbs[sub]
            outb = outbs[sub]

            @pl.when(t >= 1)
            def _drain_wb():
                pltpu.make_async_copy(
                    outb, out.at[pl.ds(0, CHUNK)], wsems[sub]).wait()

            @pl.when(ok)
            def _linear_path():
                q0 = (lo - glo) // GR
                q1 = (hi - glo) // GR
                gbase = base + glo

                def _rd(q, cc):
                    pltpu.async_copy(
                        xflat.at[pl.ds(gbase + q * GR, GR)],
                        srcb.at[pl.ds(q * GR, GR)], rsems[sub])
                    return cc
                lax.fori_loop(q0, q1 + 1, _rd, 0)

                def _wt(q, cc):
                    pltpu.make_async_copy(
                        xflat.at[pl.ds(0, GR)],
                        srcb.at[pl.ds(0, GR)], rsems[sub]).wait()
                    return cc
                lax.fori_loop(q0, q1 + 1, _wt, 0)

                off = base + glo

                def _expandv(vr, cc):
                    slots = idxb[c, pl.ds(vr * L, L)] - off
                    for l in range(L):
                        s = slots[l]
                        r = vr * L + l
                        for w in range(D // L):
                            outb[r, pl.ds(w * L, L)] = srcb[s, pl.ds(w * L, L)]
                    return cc
                lax.fori_loop(0, CHUNK // L, _expandv, 0)

            @pl.when(jnp.logical_not(ok))
            def _indirect_path():
                pltpu.async_copy(
                    xflat.at[idxb.at[c]], outb, fsems[sub]).wait()

            pltpu.async_copy(
                outb, out.at[pl.ds(b * ML + c * CHUNK, CHUNK)], wsems[sub])
        return carry
    lax.fori_loop(0, NCHUNK // 2, _pair, 0)

    for sub in range(2):
        pltpu.make_async_copy(
            outbs[sub], out.at[pl.ds(0, CHUNK)], wsems[sub]).wait()


def _expand(xflat, dur):
    mesh = plsc.VectorSubcoreMesh(core_axis_name="c", subcore_axis_name="s")
    return pl.kernel(
        _expand_body,
        out_type=jax.ShapeDtypeStruct((B * ML, D), jnp.float32),
        mesh=mesh,
        compiler_params=pltpu.CompilerParams(needs_layout_passes=False),
        scratch_types=[
            pltpu.VMEM((T + L,), jnp.int32),
            pltpu.VMEM((T,), jnp.int32),
            pltpu.VMEM((ML,), jnp.int32),
            pltpu.VMEM((ML // CHUNK, CHUNK), jnp.int32),
            [pltpu.VMEM((SRC_ROWS, D), jnp.float32) for _ in range(2)],
            [pltpu.VMEM((CHUNK, D), jnp.float32) for _ in range(2)],
            [pltpu.SemaphoreType.DMA for _ in range(2)],
            [pltpu.SemaphoreType.DMA for _ in range(2)],
            [pltpu.SemaphoreType.DMA for _ in range(2)],
        ],
    )(xflat, dur)


def _ln(h, g, be):
    mu = jnp.mean(h, axis=-1, keepdims=True)
    d = h - mu
    var = jnp.mean(d * d, axis=-1, keepdims=True)
    return d * lax.rsqrt(var + 1e-5) * g + be


def _taps(x):
    z = jnp.zeros((1, x.shape[1]), x.dtype)
    return jnp.concatenate(
        [jnp.concatenate([z, x[:-1]], 0), x, jnp.concatenate([x[1:], z], 0)],
        axis=1,
    )


def _pred_body(x_ref, m_ref, dur_ref, w1_ref, b1_ref, g1_ref, be1_ref,
               w2_ref, b2_ref, g2_ref, be2_ref, wl_ref, bl_ref,
               ld_ref, mel_ref):
    x = x_ref[0]
    h = jnp.dot(_taps(x), w1_ref[...], preferred_element_type=jnp.float32)
    h = jnp.maximum(h + b1_ref[...], 0.0)
    h = _ln(h, g1_ref[...], be1_ref[...])
    h = jnp.dot(_taps(h), w2_ref[...], preferred_element_type=jnp.float32)
    h = jnp.maximum(h + b2_ref[...], 0.0)
    h = _ln(h, g2_ref[...], be2_ref[...])
    ld = jnp.sum(h * wl_ref[...], axis=-1) + bl_ref[0, 0]
    ld_ref[0, 0, :] = ld * (1.0 - m_ref[0, 0, :])
    mel_ref[0, 0, 0] = jnp.sum(dur_ref[0, 0, :])


def _predict(x, mask_f, dur, w1r, b1r, g1r, be1r, w2r, b2r, g2r, be2r, wlr, blr):
    row3 = lambda i: (i, 0, 0)
    full = lambda i: (0, 0)
    return pl.pallas_call(
        _pred_body,
        grid=(B,),
        in_specs=[
            pl.BlockSpec((1, T, D), row3),
            pl.BlockSpec((1, 1, T), row3),
            pl.BlockSpec((1, 1, T), row3),
            pl.BlockSpec((3 * D, F), full),
            pl.BlockSpec((1, F), full),
            pl.BlockSpec((1, F), full),
            pl.BlockSpec((1, F), full),
            pl.BlockSpec((3 * F, F), full),
            pl.BlockSpec((1, F), full),
            pl.BlockSpec((1, F), full),
            pl.BlockSpec((1, F), full),
            pl.BlockSpec((1, F), full),
            pl.BlockSpec((1, 1), full),
        ],
        out_specs=[
            pl.BlockSpec((1, 1, T), row3),
            pl.BlockSpec((1, 1, 1), row3, memory_space=pltpu.SMEM),
        ],
        out_shape=[
            jax.ShapeDtypeStruct((B, 1, T), jnp.float32),
            jax.ShapeDtypeStruct((B, 1, 1), jnp.int32),
        ],
    )(x, mask_f, dur, w1r, b1r, g1r, be1r, w2r, b2r, g2r, be2r, wlr, blr)


def kernel(x, src_mask, duration, max_len, w1, b1, g1, be1, w2, b2, g2, be2, wl, bl):
    dur = duration.astype(jnp.int32)
    mask_f = src_mask.astype(jnp.float32)
    xflat = jnp.pad(x, ((0, 0), (0, TPP - T), (0, 0))).reshape(B * TPP, D)

    log_dur, mel = _predict(
        x, mask_f.reshape(B, 1, T), dur.reshape(B, 1, T),
        w1.reshape(3 * D, F), b1.reshape(1, F), g1.reshape(1, F),
        be1.reshape(1, F),
        w2.reshape(3 * F, F), b2.reshape(1, F), g2.reshape(1, F),
        be2.reshape(1, F),
        wl.reshape(1, F), bl.reshape(1, 1).astype(jnp.float32),
    )
    expanded = _expand(xflat, dur).reshape(B, ML, D)
    return (expanded, log_dur.reshape(B, T),
            mel.reshape(B).astype(duration.dtype))


# speedup vs baseline: 2.2941x; 2.2941x over previous
import functools

import jax
import jax.numpy as jnp
from jax import lax
from jax.experimental import pallas as pl
from jax.experimental.pallas import tpu as pltpu
from jax.experimental.pallas import tpu_sc as plsc

B, T, D, F = 16, 512, 256, 256
ML = 4096
TP = T + 1
TPP = 528
L = 16
NC, NS = 2, 16
NW = NC * NS
POS_PER_W = ML * B // NW
CHUNK = 64
NCHUNK = POS_PER_W // CHUNK
SRC_ROWS = 128
GR = 16


def _expand_body(xflat, dur, out, durv, cumv, zv, idxb, srcbs, outbs, zrows,
                 rsems, wsems, fsems):
    cid = lax.axis_index("c")
    sid = lax.axis_index("s")
    b = sid
    half = cid

    pltpu.sync_copy(dur.at[b], durv.at[pl.ds(0, T)])
    durv[pl.ds(T, L)] = jnp.ones((L,), jnp.int32)

    carry = jnp.int32(0)
    for j in range(T // L):
        d = durv[pl.ds(j * L, L)]
        cs = plsc.cumsum(d) + carry
        cumv[pl.ds(j * L, L)] = cs
        carry = jnp.max(cs)

    def _zero(i, c):
        zv[pl.ds(i * L, L)] = jnp.zeros((L,), jnp.int32)
        return c
    lax.fori_loop(0, ML // L, _zero, 0)

    for j in range(T // L):
        cs = cumv[pl.ds(j * L, L)]
        dnext = durv[pl.ds(j * L + 1, L)]
        val = lax.iota(jnp.int32, L) + (j * L + 1)
        m = (dnext != 0) & (cs < ML)
        plsc.store_scatter(zv, [jnp.clip(cs, 0, ML - 1)], val, mask=m)

    base = b * TPP

    def _scan_chunk(c, run):
        for v in range(CHUNK // L):
            zz = zv[pl.ds(c * CHUNK + v * L, L)]
            r = jnp.maximum(plsc.cummax(zz), run)
            idxb[c, pl.ds(v * L, L)] = r + base
            run = jnp.max(r)
        return run
    lax.fori_loop(0, ML // CHUNK, _scan_chunk, jnp.int32(0))

    def _zinit(r, cc):
        for w in range(D // L):
            zrows[r, pl.ds(w * L, L)] = jnp.zeros((L,), jnp.float32)
        return cc
    lax.fori_loop(0, CHUNK, _zinit, 0)

    def _pair(t, carry):
        meta = []
        for sub in range(2):
            m = 2 * t + sub
            c = 2 * m + half
            v0 = idxb[c, pl.ds(0, L)]
            vl = idxb[c, pl.ds(CHUNK - L, L)]
            lo = jnp.min(v0) - base
            hi = jnp.max(vl) - base
            glo = (lo // GR) * GR
            allz = lo >= T
            ok = ((hi - glo) <= (SRC_ROWS - 1)) & jnp.logical_not(allz)
            meta.append((c, lo, hi, glo, ok, allz))

            @pl.when(t >= 1)
            def _drain_wb():
                pltpu.make_async_copy(
                    outbs[sub], out.at[pl.ds(0, CHUNK)], wsems[sub]).wait()

            @pl.when(ok)
            def _issue_reads():
                gbase = base + glo

                def _rd(q, cc):
                    pltpu.async_copy(
                        xflat.at[pl.ds(gbase + q * GR, GR)],
                        srcbs[sub].at[pl.ds(q * GR, GR)], rsems[sub])
                    return cc
                lax.fori_loop((lo - glo) // GR, (hi - glo) // GR + 1, _rd, 0)

        for sub in range(2):
            c, lo, hi, glo, ok, allz = meta[sub]
            srcb = srcbs[sub]
            outb = outbs[sub]

            @pl.when(ok)
            def _linear_path():
                def _wt(q, cc):
                    pltpu.make_async_copy(
                        xflat.at[pl.ds(0, GR)],
                        srcb.at[pl.ds(0, GR)], rsems[sub]).wait()
                    return cc
                lax.fori_loop((lo - glo) // GR, (hi - glo) // GR + 1, _wt, 0)

                off = base + glo
                for v in range(CHUNK // L):
                    slots = idxb[c, pl.ds(v * L, L)] - off
                    for l in range(L):
                        s = slots[l]
                        r = v * L + l
                        for w in range(D // L):
                            outb[r, pl.ds(w * L, L)] = srcb[s, pl.ds(w * L, L)]

            @pl.when(jnp.logical_not(ok) & jnp.logical_not(allz))
            def _indirect_path():
                pltpu.async_copy(
                    xflat.at[idxb.at[c]], outb, fsems[sub]).wait()

            dst = out.at[pl.ds(b * ML + c * CHUNK, CHUNK)]

            @pl.when(allz)
            def _wb_zero():
                pltpu.async_copy(zrows, dst, wsems[sub])

            @pl.when(jnp.logical_not(allz))
            def _wb_data():
                pltpu.async_copy(outb, dst, wsems[sub])
        return carry
    lax.fori_loop(0, NCHUNK // 2, _pair, 0)

    for sub in range(2):
        pltpu.make_async_copy(
            outbs[sub], out.at[pl.ds(0, CHUNK)], wsems[sub]).wait()


def _expand(xflat, dur):
    mesh = plsc.VectorSubcoreMesh(core_axis_name="c", subcore_axis_name="s")
    return pl.kernel(
        _expand_body,
        out_type=jax.ShapeDtypeStruct((B * ML, D), jnp.float32),
        mesh=mesh,
        compiler_params=pltpu.CompilerParams(needs_layout_passes=False),
        scratch_types=[
            pltpu.VMEM((T + L,), jnp.int32),
            pltpu.VMEM((T,), jnp.int32),
            pltpu.VMEM((ML,), jnp.int32),
            pltpu.VMEM((ML // CHUNK, CHUNK), jnp.int32),
            [pltpu.VMEM((SRC_ROWS, D), jnp.float32) for _ in range(2)],
            [pltpu.VMEM((CHUNK, D), jnp.float32) for _ in range(2)],
            pltpu.VMEM((CHUNK, D), jnp.float32),
            [pltpu.SemaphoreType.DMA for _ in range(2)],
            [pltpu.SemaphoreType.DMA for _ in range(2)],
            [pltpu.SemaphoreType.DMA for _ in range(2)],
        ],
    )(xflat, dur)


def _ln(h, g, be):
    mu = jnp.mean(h, axis=-1, keepdims=True)
    d = h - mu
    var = jnp.mean(d * d, axis=-1, keepdims=True)
    return d * lax.rsqrt(var + 1e-5) * g + be


def _taps(x):
    z = jnp.zeros((1, x.shape[1]), x.dtype)
    return jnp.concatenate(
        [jnp.concatenate([z, x[:-1]], 0), x, jnp.concatenate([x[1:], z], 0)],
        axis=1,
    )


def _pred_body(x_ref, m_ref, dur_ref, w1_ref, b1_ref, g1_ref, be1_ref,
               w2_ref, b2_ref, g2_ref, be2_ref, wl_ref, bl_ref,
               ld_ref, mel_ref):
    x = x_ref[0]
    h = jnp.dot(_taps(x), w1_ref[...], preferred_element_type=jnp.float32)
    h = jnp.maximum(h + b1_ref[...], 0.0)
    h = _ln(h, g1_ref[...], be1_ref[...])
    h = jnp.dot(_taps(h), w2_ref[...], preferred_element_type=jnp.float32)
    h = jnp.maximum(h + b2_ref[...], 0.0)
    h = _ln(h, g2_ref[...], be2_ref[...])
    ld = jnp.sum(h * wl_ref[...], axis=-1) + bl_ref[0, 0]
    ld_ref[0, 0, :] = ld * (1.0 - m_ref[0, 0, :])
    mel_ref[0, 0, 0] = jnp.sum(dur_ref[0, 0, :])


def _predict(x, mask_f, dur, w1r, b1r, g1r, be1r, w2r, b2r, g2r, be2r, wlr, blr):
    row3 = lambda i: (i, 0, 0)
    full = lambda i: (0, 0)
    return pl.pallas_call(
        _pred_body,
        grid=(B,),
        in_specs=[
            pl.BlockSpec((1, T, D), row3),
            pl.BlockSpec((1, 1, T), row3),
            pl.BlockSpec((1, 1, T), row3),
            pl.BlockSpec((3 * D, F), full),
            pl.BlockSpec((1, F), full),
            pl.BlockSpec((1, F), full),
            pl.BlockSpec((1, F), full),
            pl.BlockSpec((3 * F, F), full),
            pl.BlockSpec((1, F), full),
            pl.BlockSpec((1, F), full),
            pl.BlockSpec((1, F), full),
            pl.BlockSpec((1, F), full),
            pl.BlockSpec((1, 1), full),
        ],
        out_specs=[
            pl.BlockSpec((1, 1, T), row3),
            pl.BlockSpec((1, 1, 1), row3, memory_space=pltpu.SMEM),
        ],
        out_shape=[
            jax.ShapeDtypeStruct((B, 1, T), jnp.float32),
            jax.ShapeDtypeStruct((B, 1, 1), jnp.int32),
        ],
    )(x, mask_f, dur, w1r, b1r, g1r, be1r, w2r, b2r, g2r, be2r, wlr, blr)


def kernel(x, src_mask, duration, max_len, w1, b1, g1, be1, w2, b2, g2, be2, wl, bl):
    dur = duration.astype(jnp.int32)
    mask_f = src_mask.astype(jnp.float32)
    xflat = jnp.pad(x, ((0, 0), (0, TPP - T), (0, 0))).reshape(B * TPP, D)

    log_dur, mel = _predict(
        x, mask_f.reshape(B, 1, T), dur.reshape(B, 1, T),
        w1.reshape(3 * D, F), b1.reshape(1, F), g1.reshape(1, F),
        be1.reshape(1, F),
        w2.reshape(3 * F, F), b2.reshape(1, F), g2.reshape(1, F),
        be2.reshape(1, F),
        wl.reshape(1, F), bl.reshape(1, 1).astype(jnp.float32),
    )
    expanded = _expand(xflat, dur).reshape(B, ML, D)
    return (expanded, log_dur.reshape(B, T),
            mel.reshape(B).astype(duration.dtype))


# speedup vs baseline: 2.3397x; 1.0199x over previous
import functools

import jax
import jax.numpy as jnp
from jax import lax
from jax.experimental import pallas as pl
from jax.experimental.pallas import tpu as pltpu
from jax.experimental.pallas import tpu_sc as plsc

B, T, D, F = 16, 512, 256, 256
ML = 4096
TP = T + 1
L = 16
NC, NS = 2, 16
NW = NC * NS
POS_PER_W = ML * B // NW
CHUNK = 64
NCHUNK = POS_PER_W // CHUNK
SRC_ROWS = 128
GR = 16


def _expand_body(xflat, dur, out, durv, cumv, zv, idxb, fidxc, srcbs, outbs,
                 zrows, rsems, wsems, fsems):
    cid = lax.axis_index("c")
    sid = lax.axis_index("s")
    b = sid
    half = cid

    pltpu.sync_copy(dur.at[b], durv.at[pl.ds(0, T)])
    durv[pl.ds(T, L)] = jnp.ones((L,), jnp.int32)

    carry = jnp.int32(0)
    for j in range(T // L):
        d = durv[pl.ds(j * L, L)]
        cs = plsc.cumsum(d) + carry
        cumv[pl.ds(j * L, L)] = cs
        carry = jnp.max(cs)

    def _zero(i, c):
        zv[pl.ds(i * L, L)] = jnp.zeros((L,), jnp.int32)
        return c
    lax.fori_loop(0, ML // L, _zero, 0)

    for j in range(T // L):
        cs = cumv[pl.ds(j * L, L)]
        dnext = durv[pl.ds(j * L + 1, L)]
        val = lax.iota(jnp.int32, L) + (j * L + 1)
        m = (dnext != 0) & (cs < ML)
        plsc.store_scatter(zv, [jnp.clip(cs, 0, ML - 1)], val, mask=m)

    base = b * T

    def _scan_chunk(c, run):
        for v in range(CHUNK // L):
            zz = zv[pl.ds(c * CHUNK + v * L, L)]
            r = jnp.maximum(plsc.cummax(zz), run)
            idxb[c, pl.ds(v * L, L)] = r + base
            run = jnp.max(r)
        return run
    lax.fori_loop(0, ML // CHUNK, _scan_chunk, jnp.int32(0))

    def _zinit(r, cc):
        for w in range(D // L):
            zrows[r, pl.ds(w * L, L)] = jnp.zeros((L,), jnp.float32)
        return cc
    lax.fori_loop(0, CHUNK, _zinit, 0)

    def _pair(t, carry):
        meta = []
        for sub in range(2):
            m = 2 * t + sub
            c = 2 * m + half
            v0 = idxb[c, pl.ds(0, L)]
            vl = idxb[c, pl.ds(CHUNK - L, L)]
            lo = jnp.min(v0) - base
            hi = jnp.max(vl) - base
            glo = (lo // GR) * GR
            allz = lo >= T
            ok = ((hi - glo) <= (SRC_ROWS - 1)) & jnp.logical_not(allz)
            meta.append((c, lo, hi, glo, ok, allz))

            @pl.when(t >= 1)
            def _drain_wb():
                pltpu.make_async_copy(
                    outbs[sub], out.at[pl.ds(0, CHUNK)], wsems[sub]).wait()

            @pl.when(ok)
            def _issue_reads():
                gbase = base + glo
                qe = (jnp.minimum(hi, T - 1) - glo) // GR + 1

                def _rd(q, cc):
                    pltpu.async_copy(
                        xflat.at[pl.ds(gbase + q * GR, GR)],
                        srcbs[sub].at[pl.ds(q * GR, GR)], rsems[sub])
                    return cc
                lax.fori_loop((lo - glo) // GR, qe, _rd, 0)

        for sub in range(2):
            c, lo, hi, glo, ok, allz = meta[sub]
            srcb = srcbs[sub]
            outb = outbs[sub]

            @pl.when(ok)
            def _linear_path():
                qe = (jnp.minimum(hi, T - 1) - glo) // GR + 1

                def _wt(q, cc):
                    pltpu.make_async_copy(
                        xflat.at[pl.ds(0, GR)],
                        srcb.at[pl.ds(0, GR)], rsems[sub]).wait()
                    return cc
                lax.fori_loop((lo - glo) // GR, qe, _wt, 0)

                @pl.when(hi >= T)
                def _zero_row():
                    zslot = T - glo
                    for w in range(D // L):
                        srcb[zslot, pl.ds(w * L, L)] = jnp.zeros(
                            (L,), jnp.float32)

                off = base + glo
                for v in range(CHUNK // L):
                    slots = idxb[c, pl.ds(v * L, L)] - off
                    for l in range(L):
                        s = slots[l]
                        r = v * L + l
                        for w in range(D // L):
                            outb[r, pl.ds(w * L, L)] = srcb[s, pl.ds(w * L, L)]

            @pl.when(jnp.logical_not(ok) & jnp.logical_not(allz))
            def _indirect_path():
                for v in range(CHUNK // L):
                    fi = idxb[c, pl.ds(v * L, L)]
                    fidxc[pl.ds(v * L, L)] = jnp.minimum(fi, base + T - 1)
                pltpu.async_copy(
                    xflat.at[fidxc], outb, fsems[sub]).wait()
                mel = jnp.max(cumv[pl.ds(T - L, L)])

                def _zf(r, cc):
                    @pl.when(c * CHUNK + r >= mel)
                    def _zr():
                        for w in range(D // L):
                            outb[r, pl.ds(w * L, L)] = jnp.zeros(
                                (L,), jnp.float32)
                    return cc
                lax.fori_loop(0, CHUNK, _zf, 0)

            dst = out.at[pl.ds(b * ML + c * CHUNK, CHUNK)]

            @pl.when(allz)
            def _wb_zero():
                pltpu.async_copy(zrows, dst, wsems[sub])

            @pl.when(jnp.logical_not(allz))
            def _wb_data():
                pltpu.async_copy(outb, dst, wsems[sub])
        return carry
    lax.fori_loop(0, NCHUNK // 2, _pair, 0)

    for sub in range(2):
        pltpu.make_async_copy(
            outbs[sub], out.at[pl.ds(0, CHUNK)], wsems[sub]).wait()


def _expand(xflat, dur):
    mesh = plsc.VectorSubcoreMesh(core_axis_name="c", subcore_axis_name="s")
    return pl.kernel(
        _expand_body,
        out_type=jax.ShapeDtypeStruct((B * ML, D), jnp.float32),
        mesh=mesh,
        compiler_params=pltpu.CompilerParams(needs_layout_passes=False),
        scratch_types=[
            pltpu.VMEM((T + L,), jnp.int32),
            pltpu.VMEM((T,), jnp.int32),
            pltpu.VMEM((ML,), jnp.int32),
            pltpu.VMEM((ML // CHUNK, CHUNK), jnp.int32),
            pltpu.VMEM((CHUNK,), jnp.int32),
            [pltpu.VMEM((SRC_ROWS, D), jnp.float32) for _ in range(2)],
            [pltpu.VMEM((CHUNK, D), jnp.float32) for _ in range(2)],
            pltpu.VMEM((CHUNK, D), jnp.float32),
            [pltpu.SemaphoreType.DMA for _ in range(2)],
            [pltpu.SemaphoreType.DMA for _ in range(2)],
            [pltpu.SemaphoreType.DMA for _ in range(2)],
        ],
    )(xflat, dur)


def _ln(h, g, be):
    mu = jnp.mean(h, axis=-1, keepdims=True)
    d = h - mu
    var = jnp.mean(d * d, axis=-1, keepdims=True)
    return d * lax.rsqrt(var + 1e-5) * g + be


def _taps(x):
    z = jnp.zeros((1, x.shape[1]), x.dtype)
    return jnp.concatenate(
        [jnp.concatenate([z, x[:-1]], 0), x, jnp.concatenate([x[1:], z], 0)],
        axis=1,
    )


def _pred_body(x_ref, m_ref, dur_ref, w1_ref, b1_ref, g1_ref, be1_ref,
               w2_ref, b2_ref, g2_ref, be2_ref, wl_ref, bl_ref,
               ld_ref, mel_ref):
    x = x_ref[0]
    h = jnp.dot(_taps(x), w1_ref[...], preferred_element_type=jnp.float32)
    h = jnp.maximum(h + b1_ref[...], 0.0)
    h = _ln(h, g1_ref[...], be1_ref[...])
    h = jnp.dot(_taps(h), w2_ref[...], preferred_element_type=jnp.float32)
    h = jnp.maximum(h + b2_ref[...], 0.0)
    h = _ln(h, g2_ref[...], be2_ref[...])
    ld = jnp.sum(h * wl_ref[...], axis=-1) + bl_ref[0, 0]
    ld_ref[0, 0, :] = ld * (1.0 - m_ref[0, 0, :])
    mel_ref[0, 0, 0] = jnp.sum(dur_ref[0, 0, :])


def _predict(x, mask_f, dur, w1r, b1r, g1r, be1r, w2r, b2r, g2r, be2r, wlr, blr):
    row3 = lambda i: (i, 0, 0)
    full = lambda i: (0, 0)
    return pl.pallas_call(
        _pred_body,
        grid=(B,),
        in_specs=[
            pl.BlockSpec((1, T, D), row3),
            pl.BlockSpec((1, 1, T), row3),
            pl.BlockSpec((1, 1, T), row3),
            pl.BlockSpec((3 * D, F), full),
            pl.BlockSpec((1, F), full),
            pl.BlockSpec((1, F), full),
            pl.BlockSpec((1, F), full),
            pl.BlockSpec((3 * F, F), full),
            pl.BlockSpec((1, F), full),
            pl.BlockSpec((1, F), full),
            pl.BlockSpec((1, F), full),
            pl.BlockSpec((1, F), full),
            pl.BlockSpec((1, 1), full),
        ],
        out_specs=[
            pl.BlockSpec((1, 1, T), row3),
            pl.BlockSpec((1, 1, 1), row3, memory_space=pltpu.SMEM),
        ],
        out_shape=[
            jax.ShapeDtypeStruct((B, 1, T), jnp.float32),
            jax.ShapeDtypeStruct((B, 1, 1), jnp.int32),
        ],
    )(x, mask_f, dur, w1r, b1r, g1r, be1r, w2r, b2r, g2r, be2r, wlr, blr)


def kernel(x, src_mask, duration, max_len, w1, b1, g1, be1, w2, b2, g2, be2, wl, bl):
    dur = duration.astype(jnp.int32)
    mask_f = src_mask.astype(jnp.float32)
    xflat = x.reshape(B * T, D)

    log_dur, mel = _predict(
        x, mask_f.reshape(B, 1, T), dur.reshape(B, 1, T),
        w1.reshape(3 * D, F), b1.reshape(1, F), g1.reshape(1, F),
        be1.reshape(1, F),
        w2.reshape(3 * F, F), b2.reshape(1, F), g2.reshape(1, F),
        be2.reshape(1, F),
        wl.reshape(1, F), bl.reshape(1, 1).astype(jnp.float32),
    )
    expanded = _expand(xflat, dur).reshape(B, ML, D)
    return (expanded, log_dur.reshape(B, T),
            mel.reshape(B).astype(duration.dtype))


# speedup vs baseline: 2.6839x; 1.1471x over previous
import functools

import jax
import jax.numpy as jnp
from jax import lax
from jax.experimental import pallas as pl
from jax.experimental.pallas import tpu as pltpu
from jax.experimental.pallas import tpu_sc as plsc

B, T, D, F = 16, 512, 256, 256
ML = 4096
TP = T + 1
L = 16
NC, NS = 2, 16
NW = NC * NS
POS_PER_W = ML * B // NW
CHUNK = 64
NCHUNK = POS_PER_W // CHUNK
SRC_ROWS = 128
GR = 16


def _expand_body(xflat, dur, out, durv, cumv, zv, idxb, fidxc, srcbs, outbs,
                 zrows, rsems, wsems, fsems):
    cid = lax.axis_index("c")
    sid = lax.axis_index("s")
    b = sid
    half = cid

    pltpu.sync_copy(dur.at[b], durv.at[pl.ds(0, T)])
    durv[pl.ds(T, L)] = jnp.ones((L,), jnp.int32)

    carry = jnp.int32(0)
    for j in range(T // L):
        d = durv[pl.ds(j * L, L)]
        cs = plsc.cumsum(d) + carry
        cumv[pl.ds(j * L, L)] = cs
        carry = jnp.max(cs)

    def _zero(i, c):
        zv[pl.ds(i * L, L)] = jnp.zeros((L,), jnp.int32)
        return c
    lax.fori_loop(0, ML // L, _zero, 0)

    for j in range(T // L):
        cs = cumv[pl.ds(j * L, L)]
        dnext = durv[pl.ds(j * L + 1, L)]
        val = lax.iota(jnp.int32, L) + (j * L + 1)
        m = (dnext != 0) & (cs < ML)
        plsc.store_scatter(zv, [jnp.clip(cs, 0, ML - 1)], val, mask=m)

    base = b * T

    def _scan_chunk(c, run):
        for v in range(CHUNK // L):
            zz = zv[pl.ds(c * CHUNK + v * L, L)]
            r = jnp.maximum(plsc.cummax(zz), run)
            idxb[c, pl.ds(v * L, L)] = r + base
            run = jnp.max(r)
        return run
    lax.fori_loop(0, ML // CHUNK, _scan_chunk, jnp.int32(0))

    def _zinit(r, cc):
        for w in range(D // L):
            zrows[r, pl.ds(w * L, L)] = jnp.zeros((L,), jnp.float32)
        return cc
    lax.fori_loop(0, CHUNK, _zinit, 0)

    def _pair(t, carry):
        meta = []
        for sub in range(2):
            m = 2 * t + sub
            c = 2 * m + half
            v0 = idxb[c, pl.ds(0, L)]
            vl = idxb[c, pl.ds(CHUNK - L, L)]
            lo = jnp.min(v0) - base
            hi = jnp.max(vl) - base
            glo = (lo // GR) * GR
            allz = lo >= T
            ok = ((hi - glo) <= (SRC_ROWS - 1)) & jnp.logical_not(allz)
            meta.append((c, lo, hi, glo, ok, allz))

            @pl.when(t >= 1)
            def _drain_wb():
                pltpu.make_async_copy(
                    outbs[sub], out.at[pl.ds(0, CHUNK)], wsems[sub]).wait()

            @pl.when(ok)
            def _issue_reads():
                gbase = base + glo
                qe = (jnp.minimum(hi, T - 1) - glo) // GR + 1

                def _rd(q, cc):
                    pltpu.async_copy(
                        xflat.at[pl.ds(gbase + q * GR, GR)],
                        srcbs[sub].at[pl.ds(q * GR, GR)], rsems[sub])
                    return cc
                lax.fori_loop((lo - glo) // GR, qe, _rd, 0)

        for sub in range(2):
            c, lo, hi, glo, ok, allz = meta[sub]
            srcb = srcbs[sub]
            outb = outbs[sub]

            @pl.when(ok)
            def _linear_path():
                qe = (jnp.minimum(hi, T - 1) - glo) // GR + 1

                def _wt(q, cc):
                    pltpu.make_async_copy(
                        xflat.at[pl.ds(0, GR)],
                        srcb.at[pl.ds(0, GR)], rsems[sub]).wait()
                    return cc
                lax.fori_loop((lo - glo) // GR, qe, _wt, 0)

                @pl.when(hi >= T)
                def _zero_row():
                    zslot = T - glo
                    for w in range(D // L):
                        srcb[zslot, pl.ds(w * L, L)] = jnp.zeros(
                            (L,), jnp.float32)

                off = base + glo
                for v in range(CHUNK // L):
                    slots = idxb[c, pl.ds(v * L, L)] - off
                    for l in range(L):
                        s = slots[l]
                        r = v * L + l
                        vals = [srcb[s, pl.ds(w * L, L)]
                                for w in range(D // L)]
                        for w in range(D // L):
                            outb[r, pl.ds(w * L, L)] = vals[w]

            @pl.when(jnp.logical_not(ok) & jnp.logical_not(allz))
            def _indirect_path():
                for v in range(CHUNK // L):
                    fi = idxb[c, pl.ds(v * L, L)]
                    fidxc[pl.ds(v * L, L)] = jnp.minimum(fi, base + T - 1)
                pltpu.async_copy(
                    xflat.at[fidxc], outb, fsems[sub]).wait()
                mel = jnp.max(cumv[pl.ds(T - L, L)])

                def _zf(r, cc):
                    @pl.when(c * CHUNK + r >= mel)
                    def _zr():
                        for w in range(D // L):
                            outb[r, pl.ds(w * L, L)] = jnp.zeros(
                                (L,), jnp.float32)
                    return cc
                lax.fori_loop(0, CHUNK, _zf, 0)

            dst = out.at[pl.ds(b * ML + c * CHUNK, CHUNK)]

            @pl.when(allz)
            def _wb_zero():
                pltpu.async_copy(zrows, dst, wsems[sub])

            @pl.when(jnp.logical_not(allz))
            def _wb_data():
                pltpu.async_copy(outb, dst, wsems[sub])
        return carry
    lax.fori_loop(0, NCHUNK // 2, _pair, 0)

    for sub in range(2):
        pltpu.make_async_copy(
            outbs[sub], out.at[pl.ds(0, CHUNK)], wsems[sub]).wait()


def _expand(xflat, dur):
    mesh = plsc.VectorSubcoreMesh(core_axis_name="c", subcore_axis_name="s")
    return pl.kernel(
        _expand_body,
        out_type=jax.ShapeDtypeStruct((B * ML, D), jnp.float32),
        mesh=mesh,
        compiler_params=pltpu.CompilerParams(needs_layout_passes=False),
        scratch_types=[
            pltpu.VMEM((T + L,), jnp.int32),
            pltpu.VMEM((T,), jnp.int32),
            pltpu.VMEM((ML,), jnp.int32),
            pltpu.VMEM((ML // CHUNK, CHUNK), jnp.int32),
            pltpu.VMEM((CHUNK,), jnp.int32),
            [pltpu.VMEM((SRC_ROWS, D), jnp.float32) for _ in range(2)],
            [pltpu.VMEM((CHUNK, D), jnp.float32) for _ in range(2)],
            pltpu.VMEM((CHUNK, D), jnp.float32),
            [pltpu.SemaphoreType.DMA for _ in range(2)],
            [pltpu.SemaphoreType.DMA for _ in range(2)],
            [pltpu.SemaphoreType.DMA for _ in range(2)],
        ],
    )(xflat, dur)


def _ln(h, g, be):
    mu = jnp.mean(h, axis=-1, keepdims=True)
    d = h - mu
    var = jnp.mean(d * d, axis=-1, keepdims=True)
    return d * lax.rsqrt(var + 1e-5) * g + be


def _taps(x):
    z = jnp.zeros((1, x.shape[1]), x.dtype)
    return jnp.concatenate(
        [jnp.concatenate([z, x[:-1]], 0), x, jnp.concatenate([x[1:], z], 0)],
        axis=1,
    )


def _pred_body(x_ref, m_ref, dur_ref, w1_ref, b1_ref, g1_ref, be1_ref,
               w2_ref, b2_ref, g2_ref, be2_ref, wl_ref, bl_ref,
               ld_ref, mel_ref):
    x = x_ref[0]
    h = jnp.dot(_taps(x), w1_ref[...], preferred_element_type=jnp.float32)
    h = jnp.maximum(h + b1_ref[...], 0.0)
    h = _ln(h, g1_ref[...], be1_ref[...])
    h = jnp.dot(_taps(h), w2_ref[...], preferred_element_type=jnp.float32)
    h = jnp.maximum(h + b2_ref[...], 0.0)
    h = _ln(h, g2_ref[...], be2_ref[...])
    ld = jnp.sum(h * wl_ref[...], axis=-1) + bl_ref[0, 0]
    ld_ref[0, 0, :] = ld * (1.0 - m_ref[0, 0, :])
    mel_ref[0, 0, 0] = jnp.sum(dur_ref[0, 0, :])


def _predict(x, mask_f, dur, w1r, b1r, g1r, be1r, w2r, b2r, g2r, be2r, wlr, blr):
    row3 = lambda i: (i, 0, 0)
    full = lambda i: (0, 0)
    return pl.pallas_call(
        _pred_body,
        grid=(B,),
        in_specs=[
            pl.BlockSpec((1, T, D), row3),
            pl.BlockSpec((1, 1, T), row3),
            pl.BlockSpec((1, 1, T), row3),
            pl.BlockSpec((3 * D, F), full),
            pl.BlockSpec((1, F), full),
            pl.BlockSpec((1, F), full),
            pl.BlockSpec((1, F), full),
            pl.BlockSpec((3 * F, F), full),
            pl.BlockSpec((1, F), full),
            pl.BlockSpec((1, F), full),
            pl.BlockSpec((1, F), full),
            pl.BlockSpec((1, F), full),
            pl.BlockSpec((1, 1), full),
        ],
        out_specs=[
            pl.BlockSpec((1, 1, T), row3),
            pl.BlockSpec((1, 1, 1), row3, memory_space=pltpu.SMEM),
        ],
        out_shape=[
            jax.ShapeDtypeStruct((B, 1, T), jnp.float32),
            jax.ShapeDtypeStruct((B, 1, 1), jnp.int32),
        ],
    )(x, mask_f, dur, w1r, b1r, g1r, be1r, w2r, b2r, g2r, be2r, wlr, blr)


def kernel(x, src_mask, duration, max_len, w1, b1, g1, be1, w2, b2, g2, be2, wl, bl):
    dur = duration.astype(jnp.int32)
    mask_f = src_mask.astype(jnp.float32)
    xflat = x.reshape(B * T, D)

    log_dur, mel = _predict(
        x, mask_f.reshape(B, 1, T), dur.reshape(B, 1, T),
        w1.reshape(3 * D, F), b1.reshape(1, F), g1.reshape(1, F),
        be1.reshape(1, F),
        w2.reshape(3 * F, F), b2.reshape(1, F), g2.reshape(1, F),
        be2.reshape(1, F),
        wl.reshape(1, F), bl.reshape(1, 1).astype(jnp.float32),
    )
    expanded = _expand(xflat, dur).reshape(B, ML, D)
    return (expanded, log_dur.reshape(B, T),
            mel.reshape(B).astype(duration.dtype))
